# Initial kernel scaffold; baseline (speedup 1.0000x reference)
import jax
import jax.numpy as jnp
from jax.experimental import pallas as pl

_N_TOK = 25088
_D = 64
_K = 512
_GRID = 7
_BLK = _N_TOK // _GRID


def _vq_body(flat_ref, cb_ref, idx_ref, q_ref, eloss_ref, counts_ref):
    flat = flat_ref[...]
    cb = cb_ref[...]
    cb_sq = jnp.sum(cb * cb, axis=1)
    flat_sq = jnp.sum(flat * flat, axis=1, keepdims=True)
    mm = jnp.dot(flat, cb.T, preferred_element_type=jnp.float32)
    dist = (flat_sq + cb_sq[None, :]) - 2.0 * mm
    idx = jnp.argmin(dist, axis=1).astype(jnp.int32)
    idx_ref[0, 0, :] = idx
    enc = (jax.lax.broadcasted_iota(jnp.int32, dist.shape, 1)
           == idx[:, None]).astype(jnp.float32)
    q = jnp.dot(enc, cb, preferred_element_type=jnp.float32)
    q_ref[...] = q
    d = q - flat
    eloss_ref[0, 0, 0] = jnp.sum(d * d)
    counts_ref[0, 0, :] = jnp.sum(enc, axis=0)


def _vq_quantize(flat, codebook):
    idx, q, eloss, counts = pl.pallas_call(
        _vq_body,
        grid=(_GRID,),
        in_specs=[
            pl.BlockSpec((_BLK, _D), lambda i: (i, 0)),
            pl.BlockSpec((_K, _D), lambda i: (0, 0)),
        ],
        out_specs=[
            pl.BlockSpec((1, 1, _BLK), lambda i: (i, 0, 0)),
            pl.BlockSpec((_BLK, _D), lambda i: (i, 0)),
            pl.BlockSpec((1, 1, 1), lambda i: (i, 0, 0)),
            pl.BlockSpec((1, 1, _K), lambda i: (i, 0, 0)),
        ],
        out_shape=[
            jax.ShapeDtypeStruct((_GRID, 1, _BLK), jnp.int32),
            jax.ShapeDtypeStruct((_N_TOK, _D), jnp.float32),
            jax.ShapeDtypeStruct((_GRID, 1, 1), jnp.float32),
            jax.ShapeDtypeStruct((_GRID, 1, _K), jnp.float32),
        ],
    )(flat, codebook)
    return idx.reshape(_N_TOK), q, jnp.sum(eloss), jnp.sum(counts, axis=(0, 1))


def _conv2d(x, w, b=None, stride=1, padding=0):
    out = jax.lax.conv_general_dilated(
        x, w, (stride, stride), [(padding, padding), (padding, padding)],
        dimension_numbers=('NCHW', 'OIHW', 'NCHW'))
    if b is not None:
        out = out + b[None, :, None, None]
    return out


def _conv_transpose2d(x, w, b, stride=2, padding=1):
    kh = w.shape[2]
    w2 = jnp.flip(w, axis=(2, 3)).transpose(1, 0, 2, 3)
    pad = kh - 1 - padding
    out = jax.lax.conv_general_dilated(
        x, w2, (1, 1), [(pad, pad), (pad, pad)], lhs_dilation=(stride, stride),
        dimension_numbers=('NCHW', 'OIHW', 'NCHW'))
    return out + b[None, :, None, None]


def _res_stack(x, p):
    for (w1, w2) in p:
        h = jax.nn.relu(x)
        h = _conv2d(h, w1, None, 1, 1)
        h = jax.nn.relu(h)
        h = _conv2d(h, w2, None, 1, 0)
        x = x + h
    return jax.nn.relu(x)


def kernel(x, enc_w1, enc_b1, enc_w2, enc_b2, enc_w3, enc_b3, er1_w1, er1_w2,
           er2_w1, er2_w2, pre_w, pre_b, codebook, dec_w1, dec_b1, dr1_w1,
           dr1_w2, dr2_w1, dr2_w2, dt1_w, dt1_b, dt2_w, dt2_b):
    z = jax.nn.relu(_conv2d(x, enc_w1, enc_b1, 2, 1))
    z = jax.nn.relu(_conv2d(z, enc_w2, enc_b2, 2, 1))
    z = _conv2d(z, enc_w3, enc_b3, 1, 1)
    z = _res_stack(z, [(er1_w1, er1_w2), (er2_w1, er2_w2)])
    z = _conv2d(z, pre_w, pre_b, 1, 0)

    inp = jnp.transpose(z, (0, 2, 3, 1))
    ishape = inp.shape
    flat = inp.reshape(-1, _D)

    idx, q, eloss_sum, counts = _vq_quantize(flat, codebook)

    loss = 0.25 * (eloss_sum / (_N_TOK * _D))
    avg = counts / _N_TOK
    perp = jnp.exp(-jnp.sum(avg * jnp.log(avg + 1e-10)))
    quantized = jnp.transpose(q.reshape(ishape), (0, 3, 1, 2))

    d = _conv2d(quantized, dec_w1, dec_b1, 1, 1)
    d = _res_stack(d, [(dr1_w1, dr1_w2), (dr2_w1, dr2_w2)])
    d = jax.nn.relu(_conv_transpose2d(d, dt1_w, dt1_b, 2, 1))
    x_recon = _conv_transpose2d(d, dt2_w, dt2_b, 2, 1)
    return loss, x_recon, perp, idx[:, None]


# speedup vs baseline: 1.0812x; 1.0812x over previous
import jax
import jax.numpy as jnp
from jax.experimental import pallas as pl

_N_TOK = 25088
_D = 64
_K = 512
_GRID = 7
_BLK = _N_TOK // _GRID


def _vq_body(flat_ref, cb_ref, idx_ref, q_ref, eloss_ref, counts_ref):
    flat = flat_ref[...]
    cb = cb_ref[...]
    cb_sq = jnp.sum(cb * cb, axis=1)
    flat_sq = jnp.sum(flat * flat, axis=1, keepdims=True)
    mm = jnp.dot(flat, cb.T, preferred_element_type=jnp.float32)
    dist = (flat_sq + cb_sq[None, :]) - 2.0 * mm
    idx = jnp.argmin(dist, axis=1).astype(jnp.int32)
    idx_ref[0, 0, :] = idx
    enc = (jax.lax.broadcasted_iota(jnp.int32, dist.shape, 1)
           == idx[:, None]).astype(jnp.float32)
    q = jnp.dot(enc, cb, preferred_element_type=jnp.float32)
    q_ref[...] = q
    d = q - flat
    eloss_ref[...] = jnp.sum(d * d).reshape(1, 1, 1)
    counts_ref[0, 0, :] = jnp.sum(enc, axis=0)


def _vq_quantize(flat, codebook):
    idx, q, eloss, counts = pl.pallas_call(
        _vq_body,
        grid=(_GRID,),
        in_specs=[
            pl.BlockSpec((_BLK, _D), lambda i: (i, 0)),
            pl.BlockSpec((_K, _D), lambda i: (0, 0)),
        ],
        out_specs=[
            pl.BlockSpec((1, 1, _BLK), lambda i: (i, 0, 0)),
            pl.BlockSpec((_BLK, _D), lambda i: (i, 0)),
            pl.BlockSpec((1, 1, 1), lambda i: (i, 0, 0)),
            pl.BlockSpec((1, 1, _K), lambda i: (i, 0, 0)),
        ],
        out_shape=[
            jax.ShapeDtypeStruct((_GRID, 1, _BLK), jnp.int32),
            jax.ShapeDtypeStruct((_N_TOK, _D), jnp.float32),
            jax.ShapeDtypeStruct((_GRID, 1, 1), jnp.float32),
            jax.ShapeDtypeStruct((_GRID, 1, _K), jnp.float32),
        ],
    )(flat, codebook)
    return idx.reshape(_N_TOK), q, jnp.sum(eloss), jnp.sum(counts, axis=(0, 1))


def _conv2d(x, w, b=None, stride=1, padding=0):
    out = jax.lax.conv_general_dilated(
        x, w, (stride, stride), [(padding, padding), (padding, padding)],
        dimension_numbers=('NCHW', 'OIHW', 'NCHW'))
    if b is not None:
        out = out + b[None, :, None, None]
    return out


def _conv_transpose2d(x, w, b, stride=2, padding=1):
    kh = w.shape[2]
    w2 = jnp.flip(w, axis=(2, 3)).transpose(1, 0, 2, 3)
    pad = kh - 1 - padding
    out = jax.lax.conv_general_dilated(
        x, w2, (1, 1), [(pad, pad), (pad, pad)], lhs_dilation=(stride, stride),
        dimension_numbers=('NCHW', 'OIHW', 'NCHW'))
    return out + b[None, :, None, None]


def _res_stack(x, p):
    for (w1, w2) in p:
        h = jax.nn.relu(x)
        h = _conv2d(h, w1, None, 1, 1)
        h = jax.nn.relu(h)
        h = _conv2d(h, w2, None, 1, 0)
        x = x + h
    return jax.nn.relu(x)


def kernel(x, enc_w1, enc_b1, enc_w2, enc_b2, enc_w3, enc_b3, er1_w1, er1_w2,
           er2_w1, er2_w2, pre_w, pre_b, codebook, dec_w1, dec_b1, dr1_w1,
           dr1_w2, dr2_w1, dr2_w2, dt1_w, dt1_b, dt2_w, dt2_b):
    z = jax.nn.relu(_conv2d(x, enc_w1, enc_b1, 2, 1))
    z = jax.nn.relu(_conv2d(z, enc_w2, enc_b2, 2, 1))
    z = _conv2d(z, enc_w3, enc_b3, 1, 1)
    z = _res_stack(z, [(er1_w1, er1_w2), (er2_w1, er2_w2)])
    z = _conv2d(z, pre_w, pre_b, 1, 0)

    inp = jnp.transpose(z, (0, 2, 3, 1))
    ishape = inp.shape
    flat = inp.reshape(-1, _D)

    idx, q, eloss_sum, counts = _vq_quantize(flat, codebook)

    loss = 0.25 * (eloss_sum / (_N_TOK * _D))
    avg = counts / _N_TOK
    perp = jnp.exp(-jnp.sum(avg * jnp.log(avg + 1e-10)))
    quantized = jnp.transpose(q.reshape(ishape), (0, 3, 1, 2))

    d = _conv2d(quantized, dec_w1, dec_b1, 1, 1)
    d = _res_stack(d, [(dr1_w1, dr1_w2), (dr2_w1, dr2_w2)])
    d = jax.nn.relu(_conv_transpose2d(d, dt1_w, dt1_b, 2, 1))
    x_recon = _conv_transpose2d(d, dt2_w, dt2_b, 2, 1)
    return loss, x_recon, perp, idx[:, None]
